# baseline (speedup 1.0000x reference)
import functools

import jax
import jax.numpy as jnp
from jax import lax
from jax.experimental import pallas as pl
from jax.experimental.pallas import tpu as pltpu
from jax.experimental.pallas import tpu_sc as plsc

N_GENES = 20000
N_DIM = 100
N_CELLS = 1024
G_BATCH = 1024

D_PAD = 128
BIAS_ROW = N_DIM

_info = plsc.get_sparse_core_info()
_NC, _NS = _info.num_cores, _info.num_subcores
_NW = _NC * _NS
_PPW = D_PAD // _NW


def _sc_gather_body(w1t_hbm, bias_hbm, idx_hbm, wt_out, idx_v, plane_v, out_v):
    wid = lax.axis_index("s") * _NC + lax.axis_index("c")
    pltpu.sync_copy(idx_hbm, idx_v)
    for k in range(_PPW):
        p = wid + _NW * k
        @pl.when(p < N_DIM)
        def _():
            pltpu.sync_copy(w1t_hbm.at[p], plane_v)
        @pl.when(p == BIAS_ROW)
        def _():
            pltpu.sync_copy(bias_hbm, plane_v)
        @pl.when(p <= BIAS_ROW)
        def _():
            for j in range(G_BATCH // 16):
                idx16 = idx_v[pl.ds(j * 16, 16)]
                out_v[pl.ds(j * 16, 16)] = plsc.load_gather(plane_v, [idx16])
            pltpu.sync_copy(out_v, wt_out.at[p])


def _sc_gather(w1t, bias1, gene_ix):
    mesh = plsc.VectorSubcoreMesh(core_axis_name="c", subcore_axis_name="s")
    fn = functools.partial(
        pl.kernel,
        mesh=mesh,
        out_type=jax.ShapeDtypeStruct((D_PAD, G_BATCH), jnp.float32),
        scratch_types=[
            pltpu.VMEM((G_BATCH,), jnp.int32),
            pltpu.VMEM((N_GENES,), jnp.float32),
            pltpu.VMEM((G_BATCH,), jnp.float32),
        ],
        compiler_params=pltpu.CompilerParams(needs_layout_passes=False),
    )(_sc_gather_body)
    return fn(w1t, bias1, gene_ix)


CB = 64


def _tc_body(emb_ref, wt_ref, out_ref):
    wt = wt_ref[...]
    x = emb_ref[...] * wt[:N_DIM, None, :]
    out_ref[...] = jnp.sum(x, axis=0) + wt[BIAS_ROW, None, :]


def kernel(cell_gene_embedding, gene_ix, weight1, bias1):
    gene_ix = gene_ix.astype(jnp.int32)
    w1t = jnp.transpose(weight1, (1, 0))
    wt = _sc_gather(w1t, bias1, gene_ix)

    emb_t = jnp.transpose(cell_gene_embedding, (2, 0, 1))
    out = pl.pallas_call(
        _tc_body,
        grid=(N_CELLS // CB,),
        in_specs=[
            pl.BlockSpec((N_DIM, CB, G_BATCH), lambda i: (0, i, 0)),
            pl.BlockSpec((D_PAD, G_BATCH), lambda i: (0, 0)),
        ],
        out_specs=pl.BlockSpec((CB, G_BATCH), lambda i: (i, 0)),
        out_shape=jax.ShapeDtypeStruct((N_CELLS, G_BATCH), jnp.float32),
        compiler_params=pltpu.CompilerParams(
            dimension_semantics=("arbitrary",),
            vmem_limit_bytes=100 * 1024 * 1024,
        ),
    )(emb_t, wt)
    return out


# speedup vs baseline: 1.0300x; 1.0300x over previous
import functools

import jax
import jax.numpy as jnp
from jax import lax
from jax.experimental import pallas as pl
from jax.experimental.pallas import tpu as pltpu
from jax.experimental.pallas import tpu_sc as plsc

N_GENES = 20000
N_DIM = 100
N_CELLS = 1024
G_BATCH = 1024

D_PAD = 128
BIAS_ROW = N_DIM

_info = plsc.get_sparse_core_info()
_NC, _NS = _info.num_cores, _info.num_subcores
_NW = _NC * _NS
_PPW = D_PAD // _NW


def _sc_gather_body(w1t_hbm, bias_hbm, idx_hbm, wt_out,
                    idx_v, p0, p1, p2, p3, out_v, s0, s1, s2, s3):
    wid = lax.axis_index("s") * _NC + lax.axis_index("c")
    planes = (p0, p1, p2, p3)
    sems = (s0, s1, s2, s3)
    for k in range(_PPW):
        p = wid + _NW * k
        @pl.when(p < N_DIM)
        def _():
            pltpu.make_async_copy(w1t_hbm.at[p], planes[k], sems[k]).start()
        @pl.when(p == BIAS_ROW)
        def _():
            pltpu.make_async_copy(bias_hbm, planes[k], sems[k]).start()
    pltpu.sync_copy(idx_hbm, idx_v)
    for k in range(_PPW):
        p = wid + _NW * k
        @pl.when(p <= BIAS_ROW)
        def _():
            pltpu.make_async_copy(w1t_hbm.at[p], planes[k], sems[k]).wait()
            for j in range(G_BATCH // 16):
                idx16 = idx_v[pl.ds(j * 16, 16)]
                out_v[pl.ds(j * 16, 16)] = plsc.load_gather(planes[k], [idx16])
            pltpu.sync_copy(out_v, wt_out.at[p])


def _sc_gather(w1t, bias1, gene_ix):
    mesh = plsc.VectorSubcoreMesh(core_axis_name="c", subcore_axis_name="s")
    fn = functools.partial(
        pl.kernel,
        mesh=mesh,
        out_type=jax.ShapeDtypeStruct((D_PAD, G_BATCH), jnp.float32),
        scratch_types=[
            pltpu.VMEM((G_BATCH,), jnp.int32),
            pltpu.VMEM((N_GENES,), jnp.float32),
            pltpu.VMEM((N_GENES,), jnp.float32),
            pltpu.VMEM((N_GENES,), jnp.float32),
            pltpu.VMEM((N_GENES,), jnp.float32),
            pltpu.VMEM((G_BATCH,), jnp.float32),
            pltpu.SemaphoreType.DMA,
            pltpu.SemaphoreType.DMA,
            pltpu.SemaphoreType.DMA,
            pltpu.SemaphoreType.DMA,
        ],
        compiler_params=pltpu.CompilerParams(needs_layout_passes=False),
    )(_sc_gather_body)
    return fn(w1t, bias1, gene_ix)


CB = 32


def _tc_body(emb_ref, wt_ref, out_ref):
    wt = wt_ref[...]
    x = emb_ref[...] * wt[:N_DIM, None, :]
    out_ref[...] = jnp.sum(x, axis=0) + wt[BIAS_ROW, None, :]


def kernel(cell_gene_embedding, gene_ix, weight1, bias1):
    gene_ix = gene_ix.astype(jnp.int32)
    w1t = jnp.transpose(weight1, (1, 0))
    wt = _sc_gather(w1t, bias1, gene_ix)

    emb_t = jnp.transpose(cell_gene_embedding, (2, 0, 1))
    out = pl.pallas_call(
        _tc_body,
        grid=(N_CELLS // CB,),
        in_specs=[
            pl.BlockSpec((N_DIM, CB, G_BATCH), lambda i: (0, i, 0)),
            pl.BlockSpec((D_PAD, G_BATCH), lambda i: (0, 0)),
        ],
        out_specs=pl.BlockSpec((CB, G_BATCH), lambda i: (i, 0)),
        out_shape=jax.ShapeDtypeStruct((N_CELLS, G_BATCH), jnp.float32),
        compiler_params=pltpu.CompilerParams(
            dimension_semantics=("arbitrary",),
            vmem_limit_bytes=100 * 1024 * 1024,
        ),
    )(emb_t, wt)
    return out


# speedup vs baseline: 1.0302x; 1.0002x over previous
import functools

import jax
import jax.numpy as jnp
from jax import lax
from jax.experimental import pallas as pl
from jax.experimental.pallas import tpu as pltpu
from jax.experimental.pallas import tpu_sc as plsc

N_GENES = 20000
N_DIM = 100
N_CELLS = 1024
G_BATCH = 1024

D_PAD = 128
BIAS_ROW = N_DIM

_info = plsc.get_sparse_core_info()
_NC, _NS = _info.num_cores, _info.num_subcores
_NW = _NC * _NS
_PPW = D_PAD // _NW


def _sc_gather_body(w1t_hbm, bias_hbm, idx_hbm, wt_out,
                    idx_v, p0, p1, p2, p3, out_v, s0, s1, s2, s3):
    wid = lax.axis_index("s") * _NC + lax.axis_index("c")
    planes = (p0, p1, p2, p3)
    sems = (s0, s1, s2, s3)
    for k in range(_PPW):
        p = wid + _NW * k
        @pl.when(p < N_DIM)
        def _():
            pltpu.make_async_copy(w1t_hbm.at[p], planes[k], sems[k]).start()
        @pl.when(p == BIAS_ROW)
        def _():
            pltpu.make_async_copy(bias_hbm, planes[k], sems[k]).start()
    pltpu.sync_copy(idx_hbm, idx_v)
    for k in range(_PPW):
        p = wid + _NW * k
        @pl.when(p <= BIAS_ROW)
        def _():
            pltpu.make_async_copy(w1t_hbm.at[p], planes[k], sems[k]).wait()
            for j in range(G_BATCH // 16):
                idx16 = idx_v[pl.ds(j * 16, 16)]
                out_v[pl.ds(j * 16, 16)] = plsc.load_gather(planes[k], [idx16])
            pltpu.sync_copy(out_v, wt_out.at[p])


def _sc_gather(w1t, bias1, gene_ix):
    mesh = plsc.VectorSubcoreMesh(core_axis_name="c", subcore_axis_name="s")
    fn = functools.partial(
        pl.kernel,
        mesh=mesh,
        out_type=jax.ShapeDtypeStruct((D_PAD, G_BATCH), jnp.float32),
        scratch_types=[
            pltpu.VMEM((G_BATCH,), jnp.int32),
            pltpu.VMEM((N_GENES,), jnp.float32),
            pltpu.VMEM((N_GENES,), jnp.float32),
            pltpu.VMEM((N_GENES,), jnp.float32),
            pltpu.VMEM((N_GENES,), jnp.float32),
            pltpu.VMEM((G_BATCH,), jnp.float32),
            pltpu.SemaphoreType.DMA,
            pltpu.SemaphoreType.DMA,
            pltpu.SemaphoreType.DMA,
            pltpu.SemaphoreType.DMA,
        ],
        compiler_params=pltpu.CompilerParams(needs_layout_passes=False),
    )(_sc_gather_body)
    return fn(w1t, bias1, gene_ix)


CB = 32


def _tc_body(emb_ref, wt_ref, out_ref):
    wt = wt_ref[...]
    x = emb_ref[...] * wt[:N_DIM, None, :]
    out_ref[...] = jnp.sum(x, axis=0) + wt[BIAS_ROW, None, :]


def kernel(cell_gene_embedding, gene_ix, weight1, bias1):
    gene_ix = gene_ix.astype(jnp.int32)
    w1t = jnp.transpose(weight1, (1, 0))
    wt = _sc_gather(w1t, bias1, gene_ix)

    emb_t = jnp.transpose(cell_gene_embedding, (2, 0, 1))
    out = pl.pallas_call(
        _tc_body,
        grid=(N_CELLS // CB,),
        in_specs=[
            pl.BlockSpec((N_DIM, CB, G_BATCH), lambda i: (0, i, 0)),
            pl.BlockSpec((D_PAD, G_BATCH), lambda i: (0, 0)),
        ],
        out_specs=pl.BlockSpec((CB, G_BATCH), lambda i: (i, 0)),
        out_shape=jax.ShapeDtypeStruct((N_CELLS, G_BATCH), jnp.float32),
        compiler_params=pltpu.CompilerParams(
            dimension_semantics=("parallel",),
            vmem_limit_bytes=100 * 1024 * 1024,
        ),
    )(emb_t, wt)
    return out
